# baseline (speedup 1.0000x reference)
import jax
import jax.numpy as jnp
from jax.experimental import pallas as pl
from jax.experimental.pallas import tpu as pltpu

B, C, H, W_IMG = 4, 96, 224, 224
PATCH = 14
D = 192
K = 10
L = 7
HEADS = 4
DH = D // HEADS
OUT = 128
GH, GW = H // PATCH, W_IMG // PATCH
P = GH * GW
PATCH_DIM = C * PATCH * PATCH

MB = 256
KB = PATCH_DIM // 7


def _embed_kernel(x_ref, w_ref, b_ref, o_ref):
    k = pl.program_id(1)
    acc = jnp.dot(x_ref[...], w_ref[...], preferred_element_type=jnp.float32)

    @pl.when(k == 0)
    def _():
        o_ref[...] = acc + b_ref[...]

    @pl.when(k != 0)
    def _():
        o_ref[...] += acc


def _net_kernel(h_ref, cen_ref, whg_ref, bhg_ref, ws_ref, wn_ref, wq_ref,
                wk_ref, wv_ref, wo_ref, l1s_ref, l1b_ref, l2s_ref, l2b_ref,
                w1_ref, b1_ref, w2_ref, b2_ref, o_ref):
    h = h_ref[0]
    cen = cen_ref[...]

    dots = jax.lax.dot_general(h, cen, (((1,), (1,)), ((), ())),
                               preferred_element_type=jnp.float32,
                               precision=jax.lax.Precision.HIGHEST)
    c2 = jnp.sum(cen * cen, axis=1)[None, :]
    d2 = c2 - 2.0 * dots
    iota = jax.lax.broadcasted_iota(jnp.int32, (P, K), 1)
    minv = jnp.min(d2, axis=1, keepdims=True)
    first = jnp.min(jnp.where(d2 <= minv, iota, K), axis=1, keepdims=True)
    a = (iota == first).astype(jnp.float32)
    cnt = jnp.maximum(jnp.sum(a, axis=0)[:, None], 1.0)

    def seg_mean(v):
        s = jax.lax.dot_general(a, v, (((0,), (0,)), ((), ())),
                                preferred_element_type=jnp.float32,
                                precision=jax.lax.Precision.HIGHEST)
        return jnp.dot(a, s / cnt, preferred_element_type=jnp.float32,
                       precision=jax.lax.Precision.HIGHEST)

    def ln(v, scale, bias):
        mu = jnp.mean(v, axis=-1, keepdims=True)
        var = jnp.mean((v - mu) ** 2, axis=-1, keepdims=True)
        return (v - mu) * jax.lax.rsqrt(var + 1e-5) * scale + bias

    h = jax.nn.relu(jnp.dot(seg_mean(h), whg_ref[...],
                            preferred_element_type=jnp.float32) + bhg_ref[...])

    for l in range(L):
        nbr = seg_mean(h)
        m = jax.nn.relu(
            jnp.dot(h, ws_ref[l], preferred_element_type=jnp.float32)
            + jnp.dot(nbr, wn_ref[l], preferred_element_type=jnp.float32))
        h = ln(h + m, l1s_ref[l], l1b_ref[l])
        q = jnp.dot(h, wq_ref[l], preferred_element_type=jnp.float32)
        kk = jnp.dot(h, wk_ref[l], preferred_element_type=jnp.float32)
        v = jnp.dot(h, wv_ref[l], preferred_element_type=jnp.float32)
        outs = []
        for hh in range(HEADS):
            qh = q[:, hh * DH:(hh + 1) * DH]
            kh = kk[:, hh * DH:(hh + 1) * DH]
            vh = v[:, hh * DH:(hh + 1) * DH]
            s = jax.lax.dot_general(qh, kh, (((1,), (1,)), ((), ())),
                                    preferred_element_type=jnp.float32)
            s = s * (1.0 / jnp.sqrt(float(DH)))
            s = s - jnp.max(s, axis=1, keepdims=True)
            e = jnp.exp(s)
            p_attn = e / jnp.sum(e, axis=1, keepdims=True)
            outs.append(jnp.dot(p_attn, vh, preferred_element_type=jnp.float32))
        o = jnp.dot(jnp.concatenate(outs, axis=1), wo_ref[l],
                    preferred_element_type=jnp.float32)
        h = ln(h + o, l2s_ref[l], l2b_ref[l])

    pooled = jnp.mean(h, axis=0, keepdims=True)
    y = jax.nn.relu(jnp.dot(pooled, w1_ref[...],
                            preferred_element_type=jnp.float32) + b1_ref[...])
    o_ref[0] = jnp.dot(y, w2_ref[...],
                       preferred_element_type=jnp.float32) + b2_ref[...]


def kernel(x, W_patch, b_patch, centers, W_hg, b_hg, W_gcn_self, W_gcn_nbr,
           W_q, W_k, W_v, W_o, ln1_scale, ln1_bias, ln2_scale, ln2_bias,
           W_lin1, b_lin1, W_lin2, b_lin2):
    patches = (x.reshape(B, C, GH, PATCH, GW, PATCH)
                .transpose(0, 2, 4, 1, 3, 5)
                .reshape(B * P, PATCH_DIM))

    h = pl.pallas_call(
        _embed_kernel,
        grid=(B * P // MB, PATCH_DIM // KB),
        in_specs=[
            pl.BlockSpec((MB, KB), lambda m, k: (m, k)),
            pl.BlockSpec((KB, D), lambda m, k: (k, 0)),
            pl.BlockSpec((1, D), lambda m, k: (0, 0)),
        ],
        out_specs=pl.BlockSpec((MB, D), lambda m, k: (m, 0)),
        out_shape=jax.ShapeDtypeStruct((B * P, D), jnp.float32),
        compiler_params=pltpu.CompilerParams(
            dimension_semantics=("parallel", "arbitrary")),
    )(patches, W_patch, b_patch.reshape(1, D))

    full = lambda s: pl.BlockSpec(s, lambda b: tuple(0 for _ in s))
    out3 = pl.pallas_call(
        _net_kernel,
        grid=(B,),
        in_specs=[
            pl.BlockSpec((1, P, D), lambda b: (b, 0, 0)),
            full((K, D)),
            full((D, D)), full((1, D)),
            full((L, D, D)), full((L, D, D)),
            full((L, D, D)), full((L, D, D)), full((L, D, D)), full((L, D, D)),
            full((L, D)), full((L, D)), full((L, D)), full((L, D)),
            full((D, D)), full((1, D)),
            full((D, OUT)), full((1, OUT)),
        ],
        out_specs=pl.BlockSpec((1, 1, OUT), lambda b: (b, 0, 0)),
        out_shape=jax.ShapeDtypeStruct((B, 1, OUT), jnp.float32),
        compiler_params=pltpu.CompilerParams(
            dimension_semantics=("parallel",)),
    )(h.reshape(B, P, D), centers, W_hg, b_hg.reshape(1, D),
      W_gcn_self, W_gcn_nbr, W_q, W_k, W_v, W_o,
      ln1_scale, ln1_bias, ln2_scale, ln2_bias,
      W_lin1, b_lin1.reshape(1, D), W_lin2, b_lin2.reshape(1, OUT))

    return out3.reshape(B, OUT)


# speedup vs baseline: 1.5144x; 1.5144x over previous
import jax
import jax.numpy as jnp
from jax.experimental import pallas as pl
from jax.experimental.pallas import tpu as pltpu

B, C, H, W_IMG = 4, 96, 224, 224
PATCH = 14
D = 192
K = 10
L = 7
HEADS = 4
DH = D // HEADS
OUT = 128
GH, GW = H // PATCH, W_IMG // PATCH
P = GH * GW
PATCH_DIM = C * PATCH * PATCH

MB = 256
KB = PATCH_DIM // 7


def _embed_kernel(x_ref, w_ref, b_ref, o_ref):
    xb = x_ref[0, :, 0]
    xb = xb.reshape(C, PATCH, GW, PATCH)
    pm = xb.transpose(2, 0, 1, 3).reshape(GW, PATCH_DIM)
    o_ref[...] = jnp.dot(pm, w_ref[...],
                         preferred_element_type=jnp.float32) + b_ref[...]


def _net_kernel(h_ref, cen_ref, whg_ref, bhg_ref, ws_ref, wn_ref, wq_ref,
                wk_ref, wv_ref, wo_ref, l1s_ref, l1b_ref, l2s_ref, l2b_ref,
                w1_ref, b1_ref, w2_ref, b2_ref, o_ref):
    h = h_ref[0]
    cen = cen_ref[...]

    dots = jax.lax.dot_general(h, cen, (((1,), (1,)), ((), ())),
                               preferred_element_type=jnp.float32,
                               precision=jax.lax.Precision.HIGHEST)
    c2 = jnp.sum(cen * cen, axis=1)[None, :]
    d2 = c2 - 2.0 * dots
    iota = jax.lax.broadcasted_iota(jnp.int32, (P, K), 1)
    minv = jnp.min(d2, axis=1, keepdims=True)
    first = jnp.min(jnp.where(d2 <= minv, iota, K), axis=1, keepdims=True)
    a = (iota == first).astype(jnp.float32)
    cnt = jnp.maximum(jnp.sum(a, axis=0)[:, None], 1.0)

    def seg_mean(v):
        s = jax.lax.dot_general(a, v, (((0,), (0,)), ((), ())),
                                preferred_element_type=jnp.float32,
                                precision=jax.lax.Precision.HIGHEST)
        return jnp.dot(a, s / cnt, preferred_element_type=jnp.float32,
                       precision=jax.lax.Precision.HIGHEST)

    def ln(v, scale, bias):
        mu = jnp.mean(v, axis=-1, keepdims=True)
        var = jnp.mean((v - mu) ** 2, axis=-1, keepdims=True)
        return (v - mu) * jax.lax.rsqrt(var + 1e-5) * scale + bias

    h = jax.nn.relu(jnp.dot(seg_mean(h), whg_ref[...],
                            preferred_element_type=jnp.float32) + bhg_ref[...])

    for l in range(L):
        nbr = seg_mean(h)
        m = jax.nn.relu(
            jnp.dot(h, ws_ref[l], preferred_element_type=jnp.float32)
            + jnp.dot(nbr, wn_ref[l], preferred_element_type=jnp.float32))
        h = ln(h + m, l1s_ref[l], l1b_ref[l])
        q = jnp.dot(h, wq_ref[l], preferred_element_type=jnp.float32)
        kk = jnp.dot(h, wk_ref[l], preferred_element_type=jnp.float32)
        v = jnp.dot(h, wv_ref[l], preferred_element_type=jnp.float32)
        outs = []
        for hh in range(HEADS):
            qh = q[:, hh * DH:(hh + 1) * DH]
            kh = kk[:, hh * DH:(hh + 1) * DH]
            vh = v[:, hh * DH:(hh + 1) * DH]
            s = jax.lax.dot_general(qh, kh, (((1,), (1,)), ((), ())),
                                    preferred_element_type=jnp.float32)
            s = s * (1.0 / jnp.sqrt(float(DH)))
            s = s - jnp.max(s, axis=1, keepdims=True)
            e = jnp.exp(s)
            p_attn = e / jnp.sum(e, axis=1, keepdims=True)
            outs.append(jnp.dot(p_attn, vh, preferred_element_type=jnp.float32))
        o = jnp.dot(jnp.concatenate(outs, axis=1), wo_ref[l],
                    preferred_element_type=jnp.float32)
        h = ln(h + o, l2s_ref[l], l2b_ref[l])

    pooled = jnp.mean(h, axis=0, keepdims=True)
    y = jax.nn.relu(jnp.dot(pooled, w1_ref[...],
                            preferred_element_type=jnp.float32) + b1_ref[...])
    o_ref[0] = jnp.dot(y, w2_ref[...],
                       preferred_element_type=jnp.float32) + b2_ref[...]


def kernel(x, W_patch, b_patch, centers, W_hg, b_hg, W_gcn_self, W_gcn_nbr,
           W_q, W_k, W_v, W_o, ln1_scale, ln1_bias, ln2_scale, ln2_bias,
           W_lin1, b_lin1, W_lin2, b_lin2):
    h = pl.pallas_call(
        _embed_kernel,
        grid=(B, GH),
        in_specs=[
            pl.BlockSpec((1, C, 1, PATCH, W_IMG), lambda b, g: (b, 0, g, 0, 0)),
            pl.BlockSpec((PATCH_DIM, D), lambda b, g: (0, 0)),
            pl.BlockSpec((1, D), lambda b, g: (0, 0)),
        ],
        out_specs=pl.BlockSpec((GW, D), lambda b, g: (b * GH + g, 0)),
        out_shape=jax.ShapeDtypeStruct((B * P, D), jnp.float32),
        compiler_params=pltpu.CompilerParams(
            dimension_semantics=("parallel", "parallel")),
    )(x.reshape(B, C, GH, PATCH, W_IMG), W_patch, b_patch.reshape(1, D))

    full = lambda s: pl.BlockSpec(s, lambda b: tuple(0 for _ in s))
    out3 = pl.pallas_call(
        _net_kernel,
        grid=(B,),
        in_specs=[
            pl.BlockSpec((1, P, D), lambda b: (b, 0, 0)),
            full((K, D)),
            full((D, D)), full((1, D)),
            full((L, D, D)), full((L, D, D)),
            full((L, D, D)), full((L, D, D)), full((L, D, D)), full((L, D, D)),
            full((L, D)), full((L, D)), full((L, D)), full((L, D)),
            full((D, D)), full((1, D)),
            full((D, OUT)), full((1, OUT)),
        ],
        out_specs=pl.BlockSpec((1, 1, OUT), lambda b: (b, 0, 0)),
        out_shape=jax.ShapeDtypeStruct((B, 1, OUT), jnp.float32),
        compiler_params=pltpu.CompilerParams(
            dimension_semantics=("parallel",)),
    )(h.reshape(B, P, D), centers, W_hg, b_hg.reshape(1, D),
      W_gcn_self, W_gcn_nbr, W_q, W_k, W_v, W_o,
      ln1_scale, ln1_bias, ln2_scale, ln2_bias,
      W_lin1, b_lin1.reshape(1, D), W_lin2, b_lin2.reshape(1, OUT))

    return out3.reshape(B, OUT)


# speedup vs baseline: 3.0870x; 2.0384x over previous
import jax
import jax.numpy as jnp
from jax.experimental import pallas as pl
from jax.experimental.pallas import tpu as pltpu

B, C, H, W_IMG = 4, 96, 224, 224
PATCH = 14
D = 192
K = 10
L = 7
HEADS = 4
DH = D // HEADS
OUT = 128
GH, GW = H // PATCH, W_IMG // PATCH
P = GH * GW
PATCH_DIM = C * PATCH * PATCH

MB = 256
KB = PATCH_DIM // 7


def _embed_kernel(x_ref, w_ref, b_ref, o_ref):
    m1 = x_ref[0, :, 0].reshape(C * PATCH, W_IMG)
    t = m1.T
    tr = t.reshape(GW, PATCH, C * PATCH)
    acc = b_ref[...]
    for j in range(PATCH):
        acc = acc + jnp.dot(tr[:, j, :], w_ref[j],
                            preferred_element_type=jnp.float32)
    o_ref[...] = acc


def _net_kernel(h_ref, cen_ref, whg_ref, bhg_ref, ws_ref, wn_ref, wq_ref,
                wk_ref, wv_ref, wo_ref, l1s_ref, l1b_ref, l2s_ref, l2b_ref,
                w1_ref, b1_ref, w2_ref, b2_ref, o_ref):
    h = h_ref[0]
    cen = cen_ref[...]

    dots = jax.lax.dot_general(h, cen, (((1,), (1,)), ((), ())),
                               preferred_element_type=jnp.float32,
                               precision=jax.lax.Precision.HIGHEST)
    c2 = jnp.sum(cen * cen, axis=1)[None, :]
    d2 = c2 - 2.0 * dots
    iota = jax.lax.broadcasted_iota(jnp.int32, (P, K), 1)
    minv = jnp.min(d2, axis=1, keepdims=True)
    first = jnp.min(jnp.where(d2 <= minv, iota, K), axis=1, keepdims=True)
    a = (iota == first).astype(jnp.float32)
    cnt = jnp.maximum(jnp.sum(a, axis=0)[:, None], 1.0)

    def seg_mean(v):
        s = jax.lax.dot_general(a, v, (((0,), (0,)), ((), ())),
                                preferred_element_type=jnp.float32,
                                precision=jax.lax.Precision.HIGHEST)
        return jnp.dot(a, s / cnt, preferred_element_type=jnp.float32,
                       precision=jax.lax.Precision.HIGHEST)

    def ln(v, scale, bias):
        mu = jnp.mean(v, axis=-1, keepdims=True)
        var = jnp.mean((v - mu) ** 2, axis=-1, keepdims=True)
        return (v - mu) * jax.lax.rsqrt(var + 1e-5) * scale + bias

    h = jax.nn.relu(jnp.dot(seg_mean(h), whg_ref[...],
                            preferred_element_type=jnp.float32) + bhg_ref[...])

    for l in range(L):
        nbr = seg_mean(h)
        m = jax.nn.relu(
            jnp.dot(h, ws_ref[l], preferred_element_type=jnp.float32)
            + jnp.dot(nbr, wn_ref[l], preferred_element_type=jnp.float32))
        h = ln(h + m, l1s_ref[l], l1b_ref[l])
        q = jnp.dot(h, wq_ref[l], preferred_element_type=jnp.float32)
        kk = jnp.dot(h, wk_ref[l], preferred_element_type=jnp.float32)
        v = jnp.dot(h, wv_ref[l], preferred_element_type=jnp.float32)
        outs = []
        for hh in range(HEADS):
            qh = q[:, hh * DH:(hh + 1) * DH]
            kh = kk[:, hh * DH:(hh + 1) * DH]
            vh = v[:, hh * DH:(hh + 1) * DH]
            s = jax.lax.dot_general(qh, kh, (((1,), (1,)), ((), ())),
                                    preferred_element_type=jnp.float32)
            s = s * (1.0 / jnp.sqrt(float(DH)))
            s = s - jnp.max(s, axis=1, keepdims=True)
            e = jnp.exp(s)
            p_attn = e / jnp.sum(e, axis=1, keepdims=True)
            outs.append(jnp.dot(p_attn, vh, preferred_element_type=jnp.float32))
        o = jnp.dot(jnp.concatenate(outs, axis=1), wo_ref[l],
                    preferred_element_type=jnp.float32)
        h = ln(h + o, l2s_ref[l], l2b_ref[l])

    pooled = jnp.mean(h, axis=0, keepdims=True)
    y = jax.nn.relu(jnp.dot(pooled, w1_ref[...],
                            preferred_element_type=jnp.float32) + b1_ref[...])
    o_ref[0] = jnp.dot(y, w2_ref[...],
                       preferred_element_type=jnp.float32) + b2_ref[...]


def kernel(x, W_patch, b_patch, centers, W_hg, b_hg, W_gcn_self, W_gcn_nbr,
           W_q, W_k, W_v, W_o, ln1_scale, ln1_bias, ln2_scale, ln2_bias,
           W_lin1, b_lin1, W_lin2, b_lin2):
    h = pl.pallas_call(
        _embed_kernel,
        grid=(B, GH),
        in_specs=[
            pl.BlockSpec((1, C, 1, PATCH, W_IMG), lambda b, g: (b, 0, g, 0, 0)),
            pl.BlockSpec((PATCH, C * PATCH, D), lambda b, g: (0, 0, 0)),
            pl.BlockSpec((1, D), lambda b, g: (0, 0)),
        ],
        out_specs=pl.BlockSpec((GW, D), lambda b, g: (b * GH + g, 0)),
        out_shape=jax.ShapeDtypeStruct((B * P, D), jnp.float32),
        compiler_params=pltpu.CompilerParams(
            dimension_semantics=("parallel", "parallel")),
    )(x.reshape(B, C, GH, PATCH, W_IMG),
      W_patch.reshape(C * PATCH, PATCH, D).transpose(1, 0, 2),
      b_patch.reshape(1, D))

    full = lambda s: pl.BlockSpec(s, lambda b: tuple(0 for _ in s))
    out3 = pl.pallas_call(
        _net_kernel,
        grid=(B,),
        in_specs=[
            pl.BlockSpec((1, P, D), lambda b: (b, 0, 0)),
            full((K, D)),
            full((D, D)), full((1, D)),
            full((L, D, D)), full((L, D, D)),
            full((L, D, D)), full((L, D, D)), full((L, D, D)), full((L, D, D)),
            full((L, D)), full((L, D)), full((L, D)), full((L, D)),
            full((D, D)), full((1, D)),
            full((D, OUT)), full((1, OUT)),
        ],
        out_specs=pl.BlockSpec((1, 1, OUT), lambda b: (b, 0, 0)),
        out_shape=jax.ShapeDtypeStruct((B, 1, OUT), jnp.float32),
        compiler_params=pltpu.CompilerParams(
            dimension_semantics=("parallel",)),
    )(h.reshape(B, P, D), centers, W_hg, b_hg.reshape(1, D),
      W_gcn_self, W_gcn_nbr, W_q, W_k, W_v, W_o,
      ln1_scale, ln1_bias, ln2_scale, ln2_bias,
      W_lin1, b_lin1.reshape(1, D), W_lin2, b_lin2.reshape(1, OUT))

    return out3.reshape(B, OUT)


# speedup vs baseline: 3.9639x; 1.2841x over previous
import jax
import jax.numpy as jnp
from jax.experimental import pallas as pl
from jax.experimental.pallas import tpu as pltpu

B, C, H, W_IMG = 4, 96, 224, 224
PATCH = 14
D = 192
K = 10
L = 7
HEADS = 4
DH = D // HEADS
OUT = 128
GH, GW = H // PATCH, W_IMG // PATCH
P = GH * GW
PATCH_DIM = C * PATCH * PATCH

MB = 256
KB = PATCH_DIM // 7


ROWS_PER_STEP = 4


def _embed_kernel(x_ref, w_ref, b_ref, o_ref):
    for gr in range(ROWS_PER_STEP):
        m1 = x_ref[0, :, gr * PATCH:(gr + 1) * PATCH, :].reshape(C * PATCH, W_IMG)
        t = m1.T
        tr = t.reshape(GW, PATCH, C * PATCH)
        acc = b_ref[...]
        for j in range(PATCH):
            acc = acc + jnp.dot(tr[:, j, :], w_ref[j],
                                preferred_element_type=jnp.float32)
        o_ref[gr * GW:(gr + 1) * GW, :] = acc


def _net_kernel(h_ref, cen_ref, whg_ref, bhg_ref, ws_ref, wn_ref, wq_ref,
                wk_ref, wv_ref, wo_ref, l1s_ref, l1b_ref, l2s_ref, l2b_ref,
                w1_ref, b1_ref, w2_ref, b2_ref, o_ref):
    h = h_ref[0]
    cen = cen_ref[...]

    dots = jax.lax.dot_general(h, cen, (((1,), (1,)), ((), ())),
                               preferred_element_type=jnp.float32,
                               precision=jax.lax.Precision.HIGHEST)
    c2 = jnp.sum(cen * cen, axis=1)[None, :]
    d2 = c2 - 2.0 * dots
    iota = jax.lax.broadcasted_iota(jnp.int32, (P, K), 1)
    minv = jnp.min(d2, axis=1, keepdims=True)
    first = jnp.min(jnp.where(d2 <= minv, iota, K), axis=1, keepdims=True)
    a = (iota == first).astype(jnp.float32)
    cnt = jnp.maximum(jnp.sum(a, axis=0)[:, None], 1.0)

    def seg_mean(v):
        s = jax.lax.dot_general(a, v, (((0,), (0,)), ((), ())),
                                preferred_element_type=jnp.float32,
                                precision=jax.lax.Precision.HIGHEST)
        return jnp.dot(a, s / cnt, preferred_element_type=jnp.float32,
                       precision=jax.lax.Precision.HIGHEST)

    def ln(v, scale, bias):
        mu = jnp.mean(v, axis=-1, keepdims=True)
        var = jnp.mean((v - mu) ** 2, axis=-1, keepdims=True)
        return (v - mu) * jax.lax.rsqrt(var + 1e-5) * scale + bias

    h = jax.nn.relu(jnp.dot(seg_mean(h), whg_ref[...],
                            preferred_element_type=jnp.float32) + bhg_ref[...])

    for l in range(L):
        nbr = seg_mean(h)
        m = jax.nn.relu(
            jnp.dot(h, ws_ref[l], preferred_element_type=jnp.float32)
            + jnp.dot(nbr, wn_ref[l], preferred_element_type=jnp.float32))
        h = ln(h + m, l1s_ref[l], l1b_ref[l])
        q = jnp.dot(h, wq_ref[l], preferred_element_type=jnp.float32)
        kk = jnp.dot(h, wk_ref[l], preferred_element_type=jnp.float32)
        v = jnp.dot(h, wv_ref[l], preferred_element_type=jnp.float32)
        outs = []
        for hh in range(HEADS):
            qh = q[:, hh * DH:(hh + 1) * DH]
            kh = kk[:, hh * DH:(hh + 1) * DH]
            vh = v[:, hh * DH:(hh + 1) * DH]
            s = jax.lax.dot_general(qh, kh, (((1,), (1,)), ((), ())),
                                    preferred_element_type=jnp.float32)
            s = s * (1.0 / jnp.sqrt(float(DH)))
            s = s - jnp.max(s, axis=1, keepdims=True)
            e = jnp.exp(s)
            p_attn = e / jnp.sum(e, axis=1, keepdims=True)
            outs.append(jnp.dot(p_attn, vh, preferred_element_type=jnp.float32))
        o = jnp.dot(jnp.concatenate(outs, axis=1), wo_ref[l],
                    preferred_element_type=jnp.float32)
        h = ln(h + o, l2s_ref[l], l2b_ref[l])

    pooled = jnp.mean(h, axis=0, keepdims=True)
    y = jax.nn.relu(jnp.dot(pooled, w1_ref[...],
                            preferred_element_type=jnp.float32) + b1_ref[...])
    o_ref[0] = jnp.dot(y, w2_ref[...],
                       preferred_element_type=jnp.float32) + b2_ref[...]


def kernel(x, W_patch, b_patch, centers, W_hg, b_hg, W_gcn_self, W_gcn_nbr,
           W_q, W_k, W_v, W_o, ln1_scale, ln1_bias, ln2_scale, ln2_bias,
           W_lin1, b_lin1, W_lin2, b_lin2):
    h = pl.pallas_call(
        _embed_kernel,
        grid=(B, GH // ROWS_PER_STEP),
        in_specs=[
            pl.BlockSpec((1, C, ROWS_PER_STEP * PATCH, W_IMG),
                         lambda b, g: (b, 0, g, 0)),
            pl.BlockSpec((PATCH, C * PATCH, D), lambda b, g: (0, 0, 0)),
            pl.BlockSpec((1, D), lambda b, g: (0, 0)),
        ],
        out_specs=pl.BlockSpec((ROWS_PER_STEP * GW, D),
                               lambda b, g: (b * (GH // ROWS_PER_STEP) + g, 0)),
        out_shape=jax.ShapeDtypeStruct((B * P, D), jnp.float32),
        compiler_params=pltpu.CompilerParams(
            dimension_semantics=("parallel", "parallel")),
    )(x,
      W_patch.reshape(C * PATCH, PATCH, D).transpose(1, 0, 2),
      b_patch.reshape(1, D))

    full = lambda s: pl.BlockSpec(s, lambda b: tuple(0 for _ in s))
    out3 = pl.pallas_call(
        _net_kernel,
        grid=(B,),
        in_specs=[
            pl.BlockSpec((1, P, D), lambda b: (b, 0, 0)),
            full((K, D)),
            full((D, D)), full((1, D)),
            full((L, D, D)), full((L, D, D)),
            full((L, D, D)), full((L, D, D)), full((L, D, D)), full((L, D, D)),
            full((L, D)), full((L, D)), full((L, D)), full((L, D)),
            full((D, D)), full((1, D)),
            full((D, OUT)), full((1, OUT)),
        ],
        out_specs=pl.BlockSpec((1, 1, OUT), lambda b: (b, 0, 0)),
        out_shape=jax.ShapeDtypeStruct((B, 1, OUT), jnp.float32),
        compiler_params=pltpu.CompilerParams(
            dimension_semantics=("parallel",)),
    )(h.reshape(B, P, D), centers, W_hg, b_hg.reshape(1, D),
      W_gcn_self, W_gcn_nbr, W_q, W_k, W_v, W_o,
      ln1_scale, ln1_bias, ln2_scale, ln2_bias,
      W_lin1, b_lin1.reshape(1, D), W_lin2, b_lin2.reshape(1, OUT))

    return out3.reshape(B, OUT)


# speedup vs baseline: 3.9922x; 1.0071x over previous
import jax
import jax.numpy as jnp
from jax.experimental import pallas as pl
from jax.experimental.pallas import tpu as pltpu

B, C, H, W_IMG = 4, 96, 224, 224
PATCH = 14
D = 192
K = 10
L = 7
HEADS = 4
DH = D // HEADS
OUT = 128
GH, GW = H // PATCH, W_IMG // PATCH
P = GH * GW
PATCH_DIM = C * PATCH * PATCH

ROWS_PER_STEP = 4
EMBED_STEPS = GH // ROWS_PER_STEP


def _fused_kernel(x_ref, w_ref, b_ref, cen_ref, whg_ref, bhg_ref, ws_ref,
                  wn_ref, wq_ref, wk_ref, wv_ref, wo_ref, l1s_ref, l1b_ref,
                  l2s_ref, l2b_ref, w1_ref, b1_ref, w2_ref, b2_ref, o_ref,
                  h_scr):
    g = pl.program_id(1)

    @pl.when(g < EMBED_STEPS)
    def _embed():
        for gr in range(ROWS_PER_STEP):
            m1 = x_ref[0, :, gr * PATCH:(gr + 1) * PATCH, :].reshape(
                C * PATCH, W_IMG)
            t = m1.T
            tr = t.reshape(GW, PATCH, C * PATCH)
            acc = b_ref[...]
            for j in range(PATCH):
                acc = acc + jnp.dot(tr[:, j, :], w_ref[j],
                                    preferred_element_type=jnp.float32)
            h_scr[pl.ds(g * (ROWS_PER_STEP * GW) + gr * GW, GW), :] = acc

    @pl.when(g == EMBED_STEPS)
    def _net():
        h = h_scr[...]
        cen = cen_ref[...]

        dots = jax.lax.dot_general(h, cen, (((1,), (1,)), ((), ())),
                                   preferred_element_type=jnp.float32,
                                   precision=jax.lax.Precision.HIGHEST)
        c2 = jnp.sum(cen * cen, axis=1)[None, :]
        d2 = c2 - 2.0 * dots
        iota = jax.lax.broadcasted_iota(jnp.int32, (P, K), 1)
        minv = jnp.min(d2, axis=1, keepdims=True)
        first = jnp.min(jnp.where(d2 <= minv, iota, K), axis=1, keepdims=True)
        a = (iota == first).astype(jnp.float32)
        cnt = jnp.maximum(jnp.sum(a, axis=0)[:, None], 1.0)

        def seg_mean(v):
            s = jax.lax.dot_general(a, v, (((0,), (0,)), ((), ())),
                                    preferred_element_type=jnp.float32,
                                    precision=jax.lax.Precision.HIGHEST)
            return jnp.dot(a, s / cnt, preferred_element_type=jnp.float32,
                           precision=jax.lax.Precision.HIGHEST)

        def ln(v, scale, bias):
            mu = jnp.mean(v, axis=-1, keepdims=True)
            var = jnp.mean((v - mu) ** 2, axis=-1, keepdims=True)
            return (v - mu) * jax.lax.rsqrt(var + 1e-5) * scale + bias

        hh0 = jax.nn.relu(jnp.dot(seg_mean(h), whg_ref[...],
                                  preferred_element_type=jnp.float32)
                          + bhg_ref[...])
        h = hh0
        for l in range(L):
            nbr = seg_mean(h)
            m = jax.nn.relu(
                jnp.dot(h, ws_ref[l], preferred_element_type=jnp.float32)
                + jnp.dot(nbr, wn_ref[l], preferred_element_type=jnp.float32))
            h = ln(h + m, l1s_ref[l], l1b_ref[l])
            q = jnp.dot(h, wq_ref[l], preferred_element_type=jnp.float32)
            kk = jnp.dot(h, wk_ref[l], preferred_element_type=jnp.float32)
            v = jnp.dot(h, wv_ref[l], preferred_element_type=jnp.float32)
            outs = []
            for hd in range(HEADS):
                qh = q[:, hd * DH:(hd + 1) * DH]
                kh = kk[:, hd * DH:(hd + 1) * DH]
                vh = v[:, hd * DH:(hd + 1) * DH]
                s = jax.lax.dot_general(qh, kh, (((1,), (1,)), ((), ())),
                                        preferred_element_type=jnp.float32)
                s = s * (1.0 / jnp.sqrt(float(DH)))
                s = s - jnp.max(s, axis=1, keepdims=True)
                e = jnp.exp(s)
                p_attn = e / jnp.sum(e, axis=1, keepdims=True)
                outs.append(jnp.dot(p_attn, vh,
                                    preferred_element_type=jnp.float32))
            o = jnp.dot(jnp.concatenate(outs, axis=1), wo_ref[l],
                        preferred_element_type=jnp.float32)
            h = ln(h + o, l2s_ref[l], l2b_ref[l])

        pooled = jnp.mean(h, axis=0, keepdims=True)
        y = jax.nn.relu(jnp.dot(pooled, w1_ref[...],
                                preferred_element_type=jnp.float32)
                        + b1_ref[...])
        o_ref[0] = jnp.dot(y, w2_ref[...],
                           preferred_element_type=jnp.float32) + b2_ref[...]


def kernel(x, W_patch, b_patch, centers, W_hg, b_hg, W_gcn_self, W_gcn_nbr,
           W_q, W_k, W_v, W_o, ln1_scale, ln1_bias, ln2_scale, ln2_bias,
           W_lin1, b_lin1, W_lin2, b_lin2):
    full = lambda s: pl.BlockSpec(s, lambda b, g: tuple(0 for _ in s))
    out3 = pl.pallas_call(
        _fused_kernel,
        grid=(B, EMBED_STEPS + 1),
        in_specs=[
            pl.BlockSpec((1, C, ROWS_PER_STEP * PATCH, W_IMG),
                         lambda b, g: (b, 0, jnp.minimum(g, EMBED_STEPS - 1), 0)),
            full((PATCH, C * PATCH, D)),
            full((1, D)),
            full((K, D)),
            full((D, D)), full((1, D)),
            full((L, D, D)), full((L, D, D)),
            full((L, D, D)), full((L, D, D)), full((L, D, D)), full((L, D, D)),
            full((L, D)), full((L, D)), full((L, D)), full((L, D)),
            full((D, D)), full((1, D)),
            full((D, OUT)), full((1, OUT)),
        ],
        out_specs=pl.BlockSpec((1, 1, OUT), lambda b, g: (b, 0, 0)),
        out_shape=jax.ShapeDtypeStruct((B, 1, OUT), jnp.float32),
        scratch_shapes=[
            pltpu.VMEM((P, D), jnp.float32),
        ],
        compiler_params=pltpu.CompilerParams(
            dimension_semantics=("arbitrary", "arbitrary")),
    )(x, W_patch.reshape(C * PATCH, PATCH, D).transpose(1, 0, 2),
      b_patch.reshape(1, D), centers, W_hg, b_hg.reshape(1, D),
      W_gcn_self, W_gcn_nbr, W_q, W_k, W_v, W_o,
      ln1_scale, ln1_bias, ln2_scale, ln2_bias,
      W_lin1, b_lin1.reshape(1, D), W_lin2, b_lin2.reshape(1, OUT))

    return out3.reshape(B, OUT)


# speedup vs baseline: 5.3133x; 1.3309x over previous
import jax
import jax.numpy as jnp
from jax.experimental import pallas as pl
from jax.experimental.pallas import tpu as pltpu

B, C, H, W_IMG = 4, 96, 224, 224
PATCH = 14
D = 192
K = 10
L = 7
HEADS = 4
DH = D // HEADS
OUT = 128
GH, GW = H // PATCH, W_IMG // PATCH
P = GH * GW
PATCH_DIM = C * PATCH * PATCH

ROWS_PER_STEP = 4
EMBED_STEPS = GH // ROWS_PER_STEP


def _fused_kernel(x_ref, w_ref, b_ref, cen_ref, whg_ref, bhg_ref, ws_ref,
                  wn_ref, wq_ref, wk_ref, wv_ref, wo_ref, l1s_ref, l1b_ref,
                  l2s_ref, l2b_ref, w1_ref, b1_ref, w2_ref, b2_ref, o_ref,
                  h_scr):
    g = pl.program_id(1)

    @pl.when(g < EMBED_STEPS)
    def _embed():
        for gr in range(ROWS_PER_STEP):
            m1 = x_ref[0, :, gr * PATCH:(gr + 1) * PATCH, :].reshape(
                C * PATCH, W_IMG)
            t = m1.T
            tr = t.reshape(GW, PATCH, C * PATCH)
            acc = b_ref[...]
            for j in range(PATCH):
                acc = acc + jnp.dot(tr[:, j, :], w_ref[j],
                                    preferred_element_type=jnp.float32)
            h_scr[pl.ds(g * (ROWS_PER_STEP * GW) + gr * GW, GW), :] = acc

    @pl.when(g == EMBED_STEPS)
    def _net():
        h = h_scr[...]
        cen = cen_ref[...]

        dots = jax.lax.dot_general(h, cen, (((1,), (1,)), ((), ())),
                                   preferred_element_type=jnp.float32,
                                   precision=jax.lax.Precision.HIGHEST)
        c2 = jnp.sum(cen * cen, axis=1)[None, :]
        d2 = c2 - 2.0 * dots
        iota = jax.lax.broadcasted_iota(jnp.int32, (P, K), 1)
        minv = jnp.min(d2, axis=1, keepdims=True)
        first = jnp.min(jnp.where(d2 <= minv, iota, K), axis=1, keepdims=True)
        a = (iota == first).astype(jnp.float32)
        cnt = jnp.maximum(jnp.sum(a, axis=0)[:, None], 1.0)

        def seg_mean(v):
            s = jax.lax.dot_general(a, v, (((0,), (0,)), ((), ())),
                                    preferred_element_type=jnp.float32,
                                    precision=jax.lax.Precision.HIGHEST)
            return jnp.dot(a, s / cnt, preferred_element_type=jnp.float32,
                           precision=jax.lax.Precision.HIGHEST)

        def ln(v, scale, bias):
            mu = jnp.mean(v, axis=-1, keepdims=True)
            var = jnp.mean((v - mu) ** 2, axis=-1, keepdims=True)
            return (v - mu) * jax.lax.rsqrt(var + 1e-5) * scale + bias

        hh0 = jax.nn.relu(jnp.dot(seg_mean(h), whg_ref[...],
                                  preferred_element_type=jnp.float32)
                          + bhg_ref[...])
        h = hh0
        for l in range(L):
            nbr = seg_mean(h)
            m = jax.nn.relu(
                jnp.dot(h, ws_ref[l], preferred_element_type=jnp.float32)
                + jnp.dot(nbr, wn_ref[l], preferred_element_type=jnp.float32))
            h = ln(h + m, l1s_ref[l], l1b_ref[l])
            q = jnp.dot(h, wq_ref[l], preferred_element_type=jnp.float32)
            kk = jnp.dot(h, wk_ref[l], preferred_element_type=jnp.float32)
            v = jnp.dot(h, wv_ref[l], preferred_element_type=jnp.float32)
            outs = []
            for hd in range(HEADS):
                qh = q[:, hd * DH:(hd + 1) * DH]
                kh = kk[:, hd * DH:(hd + 1) * DH]
                vh = v[:, hd * DH:(hd + 1) * DH]
                s = jax.lax.dot_general(qh, kh, (((1,), (1,)), ((), ())),
                                        preferred_element_type=jnp.float32)
                s = s * (1.0 / jnp.sqrt(float(DH)))
                s = s - jnp.max(s, axis=1, keepdims=True)
                e = jnp.exp(s)
                p_attn = e / jnp.sum(e, axis=1, keepdims=True)
                outs.append(jnp.dot(p_attn, vh,
                                    preferred_element_type=jnp.float32))
            o = jnp.dot(jnp.concatenate(outs, axis=1), wo_ref[l],
                        preferred_element_type=jnp.float32)
            h = ln(h + o, l2s_ref[l], l2b_ref[l])

        pooled = jnp.mean(h, axis=0, keepdims=True)
        y = jax.nn.relu(jnp.dot(pooled, w1_ref[...],
                                preferred_element_type=jnp.float32)
                        + b1_ref[...])
        o_ref[0] = jnp.dot(y, w2_ref[...],
                           preferred_element_type=jnp.float32) + b2_ref[...]


WCHUNK = 192


def _wperm_kernel(w_ref, o_ref):
    o_ref[...] = w_ref[...].reshape(WCHUNK, PATCH, D).transpose(1, 0, 2)


def kernel(x, W_patch, b_patch, centers, W_hg, b_hg, W_gcn_self, W_gcn_nbr,
           W_q, W_k, W_v, W_o, ln1_scale, ln1_bias, ln2_scale, ln2_bias,
           W_lin1, b_lin1, W_lin2, b_lin2):
    wp = pl.pallas_call(
        _wperm_kernel,
        grid=(C * PATCH // WCHUNK,),
        in_specs=[pl.BlockSpec((WCHUNK * PATCH, D), lambda i: (i, 0))],
        out_specs=pl.BlockSpec((PATCH, WCHUNK, D), lambda i: (0, i, 0)),
        out_shape=jax.ShapeDtypeStruct((PATCH, C * PATCH, D), jnp.float32),
        compiler_params=pltpu.CompilerParams(
            dimension_semantics=("arbitrary",)),
    )(W_patch)

    full = lambda s: pl.BlockSpec(s, lambda b, g: tuple(0 for _ in s))
    out3 = pl.pallas_call(
        _fused_kernel,
        grid=(B, EMBED_STEPS + 1),
        in_specs=[
            pl.BlockSpec((1, C, ROWS_PER_STEP * PATCH, W_IMG),
                         lambda b, g: (b, 0, jnp.minimum(g, EMBED_STEPS - 1), 0)),
            full((PATCH, C * PATCH, D)),
            full((1, D)),
            full((K, D)),
            full((D, D)), full((1, D)),
            full((L, D, D)), full((L, D, D)),
            full((L, D, D)), full((L, D, D)), full((L, D, D)), full((L, D, D)),
            full((L, D)), full((L, D)), full((L, D)), full((L, D)),
            full((D, D)), full((1, D)),
            full((D, OUT)), full((1, OUT)),
        ],
        out_specs=pl.BlockSpec((1, 1, OUT), lambda b, g: (b, 0, 0)),
        out_shape=jax.ShapeDtypeStruct((B, 1, OUT), jnp.float32),
        scratch_shapes=[
            pltpu.VMEM((P, D), jnp.float32),
        ],
        compiler_params=pltpu.CompilerParams(
            dimension_semantics=("arbitrary", "arbitrary")),
    )(x, wp, b_patch.reshape(1, D), centers, W_hg, b_hg.reshape(1, D),
      W_gcn_self, W_gcn_nbr, W_q, W_k, W_v, W_o,
      ln1_scale, ln1_bias, ln2_scale, ln2_bias,
      W_lin1, b_lin1.reshape(1, D), W_lin2, b_lin2.reshape(1, OUT))

    return out3.reshape(B, OUT)
